# Initial kernel scaffold; baseline (speedup 1.0000x reference)
import jax
import jax.numpy as jnp
from jax.experimental import pallas as pl


def kernel(x, batch):
    raise NotImplementedError("write your pallas kernel here")


# speedup vs baseline: 1.0011x; 1.0011x over previous
import jax
import jax.numpy as jnp
from jax.experimental import pallas as pl
from jax.experimental.pallas import tpu as pltpu

_NOISE = 0.001
_SCALE = -1.0 / _NOISE


def _nll_kernel(x_ref, b_ref, o_ref):
    b = pl.program_id(0)
    t = pl.program_id(1)
    xv = x_ref[...]
    bv = b_ref[...]
    d = jnp.where(jnp.isfinite(bv), bv - xv, 0.0)
    s = _SCALE * jnp.sum(d * d)

    @pl.when(t == 0)
    def _init():
        o_ref[b] = s

    @pl.when(t != 0)
    def _acc():
        o_ref[b] += s


def kernel(x, batch):
    nb, nt, h, w = x.shape
    x2 = x.reshape(nb, nt * h, w)
    b2 = batch.reshape(nb, nt * h, w)
    chunk = 2048
    n_chunks = (nt * h) // chunk

    out = pl.pallas_call(
        _nll_kernel,
        grid=(nb, n_chunks),
        in_specs=[
            pl.BlockSpec((1, chunk, w), lambda b, t: (b, t, 0)),
            pl.BlockSpec((1, chunk, w), lambda b, t: (b, t, 0)),
        ],
        out_specs=pl.BlockSpec(
            (nb,), lambda b, t: (0,), memory_space=pltpu.SMEM
        ),
        out_shape=jax.ShapeDtypeStruct((nb,), jnp.float32),
    )(x2, b2)
    return out
